# baseline (speedup 1.0000x reference)
import functools

import numpy as np
import jax
import jax.numpy as jnp
from jax import lax
from jax.experimental import pallas as pl
from jax.experimental.pallas import tpu as pltpu
from jax.experimental.pallas import tpu_sc as plsc

NUM_CORES = 2
NUM_SUBCORES = 16
NW = NUM_CORES * NUM_SUBCORES

BATCH = 16384
NUM_EMB = 1000000
HIST = 50
HIST_PAD = 56
GROUP = 4
IDX_PER_DMA = HIST * GROUP
DIM = 32
RING = 4

ROWS_PER_W = BATCH // NW
GROUPS_PER_W = ROWS_PER_W // GROUP

CHUNK = 128
FULL_CHUNKS = NUM_EMB // CHUNK
TAIL = NUM_EMB - FULL_CHUNKS * CHUNK
BASE_CHUNKS = FULL_CHUNKS // NW
EXTRA = FULL_CHUNKS - BASE_CHUNKS * NW

_RSQRT_TAB = np.zeros((64,), np.float32)
_RSQRT_TAB[: HIST + 1] = (
    1.0 / np.sqrt(np.maximum(np.arange(HIST + 1, dtype=np.float64), 1.0))
).astype(np.float32)


def _transpose_body(w_t3_hbm, w_tail_hbm, out_hbm,
                    tbuf0, tbuf1, obuf0, obuf1, tail_v, si0, si1, so0, so1):
    tbufs = (tbuf0, tbuf1)
    obufs = (obuf0, obuf1)
    sin = (si0, si1)
    sout = (so0, so1)
    wid = lax.axis_index("s") * NUM_CORES + lax.axis_index("c")
    n_w = BASE_CHUNKS + (wid < EXTRA).astype(jnp.int32)
    lo_w = wid * BASE_CHUNKS + jnp.minimum(wid, EXTRA)

    @pl.when(wid == 0)
    def _():
        pltpu.sync_copy(w_tail_hbm, tail_v)
        pltpu.sync_copy(tail_v, out_hbm.at[pl.ds(FULL_CHUNKS * CHUNK * DIM,
                                                 TAIL * DIM)])

    def start_in(c, s):
        pltpu.make_async_copy(
            w_t3_hbm.at[:, :, pl.ds(c * CHUNK, CHUNK)], tbufs[s], sin[s]
        ).start()

    def wait_in(s):
        pltpu.make_async_copy(
            w_t3_hbm.at[:, :, pl.ds(0, CHUNK)], tbufs[s], sin[s]
        ).wait()

    def start_out(c, s):
        pltpu.make_async_copy(
            obufs[s], out_hbm.at[pl.ds(c * CHUNK * DIM, CHUNK * DIM)],
            sout[s]
        ).start()

    def wait_out(s):
        pltpu.make_async_copy(
            obufs[s], out_hbm.at[pl.ds(0, CHUNK * DIM)], sout[s]
        ).wait()

    lane32 = lax.iota(jnp.int32, 16) * DIM

    def transpose_chunk(s):
        for rb in range(4):
            for d in range(8):
                f = 8 * rb + d
                for v in range(CHUNK // 16):
                    vec = tbufs[s][rb, d, pl.ds(16 * v, 16)]
                    plsc.store_scatter(
                        obufs[s], [lane32 + (16 * v * DIM + f)], vec
                    )

    start_in(lo_w, 0)
    start_in(lo_w + 1, 1)

    def loop_body(i, carry):
        for s in range(2):
            k = 2 * i + s
            c = lo_w + k

            @pl.when(k < n_w)
            def _():
                wait_in(s)

                @pl.when(k >= 2)
                def _():
                    wait_out(s)

                transpose_chunk(s)
                start_out(c, s)

                @pl.when(k + 2 < n_w)
                def _():
                    start_in(c + 2, s)

        return carry

    lax.fori_loop(0, (BASE_CHUNKS + 2) // 2, loop_body, 0)

    wait_out(0)
    wait_out(1)


_transpose = functools.partial(
    pl.kernel,
    out_type=jax.ShapeDtypeStruct((NUM_EMB * DIM,), jnp.float32),
    mesh=plsc.VectorSubcoreMesh(core_axis_name="c", subcore_axis_name="s"),
    compiler_params=pltpu.CompilerParams(
        use_tc_tiling_on_sc=True, needs_layout_passes=False
    ),
    scratch_types=[
        pltpu.VMEM((4, 8, CHUNK), jnp.float32),
        pltpu.VMEM((4, 8, CHUNK), jnp.float32),
        pltpu.VMEM((CHUNK * DIM,), jnp.float32),
        pltpu.VMEM((CHUNK * DIM,), jnp.float32),
        pltpu.VMEM((TAIL * DIM,), jnp.float32),
        pltpu.SemaphoreType.DMA,
        pltpu.SemaphoreType.DMA,
        pltpu.SemaphoreType.DMA,
        pltpu.SemaphoreType.DMA,
    ],
)(_transpose_body)


def _emb_bag_body(idx_g_hbm, idx_c_hbm, table_hbm, rtab_hbm, out_hbm,
                  idx_v, idx_c_v, gbuf, out_v, rtab_v, s0, s1, s2, s3):
    sems = (s0, s1, s2, s3)
    wid = lax.axis_index("s") * NUM_CORES + lax.axis_index("c")
    group_base = wid * GROUPS_PER_W
    row_base = wid * ROWS_PER_W

    pltpu.sync_copy(idx_g_hbm.at[pl.ds(group_base, GROUPS_PER_W)], idx_v)
    pltpu.sync_copy(idx_c_hbm.at[pl.ds(row_base, ROWS_PER_W)], idx_c_v)
    pltpu.sync_copy(rtab_hbm, rtab_v)

    def start(p, b):
        pltpu.make_async_copy(
            table_hbm.at[idx_v.at[p]], gbuf.at[b], sems[b]
        ).start()

    def wait(b):
        pltpu.make_async_copy(
            table_hbm.at[idx_v.at[0]], gbuf.at[b], sems[b]
        ).wait()

    def compute(p, b):
        for r in range(GROUP):
            out_row = p * GROUP + r
            i0 = idx_c_v[out_row, pl.ds(0, 16)]
            i1 = idx_c_v[out_row, pl.ds(16, 16)]
            i2 = idx_c_v[out_row, pl.ds(32, 16)]
            i3 = idx_c_v[out_row, pl.ds(40, 16)]
            tail_m = lax.iota(jnp.int32, 16) >= 8
            c_vec = (
                plsc.all_reduce_population_count(i0 != 0)
                + plsc.all_reduce_population_count(i1 != 0)
                + plsc.all_reduce_population_count(i2 != 0)
                + plsc.all_reduce_population_count((i3 != 0) & tail_m)
            )
            scale = plsc.load_gather(rtab_v, [c_vec])

            acc = [jnp.zeros((16,), jnp.float32) for _ in range(4)]
            for l in range(HIST):
                row = r * HIST + l
                j = l & 1
                acc[j] = acc[j] + gbuf[b, row, pl.ds(0, 16)]
                acc[2 + j] = acc[2 + j] + gbuf[b, row, pl.ds(16, 16)]

            out_v[out_row, pl.ds(0, 16)] = (acc[0] + acc[1]) * scale
            out_v[out_row, pl.ds(16, 16)] = (acc[2] + acc[3]) * scale

    for b in range(RING):
        start(b, b)

    def loop_body(i, carry):
        p0 = i * RING
        for b in range(RING):
            p = p0 + b
            wait(b)
            compute(p, b)

            @pl.when(p + RING < GROUPS_PER_W)
            def _():
                start(p + RING, b)

        return carry

    lax.fori_loop(0, GROUPS_PER_W // RING, loop_body, 0)

    pltpu.sync_copy(out_v, out_hbm.at[pl.ds(row_base, ROWS_PER_W)])


_emb_bag = functools.partial(
    pl.kernel,
    out_type=jax.ShapeDtypeStruct((BATCH, DIM), jnp.float32),
    mesh=plsc.VectorSubcoreMesh(core_axis_name="c", subcore_axis_name="s"),
    compiler_params=pltpu.CompilerParams(
        use_tc_tiling_on_sc=False, needs_layout_passes=False
    ),
    scratch_types=[
        pltpu.VMEM((GROUPS_PER_W, IDX_PER_DMA), jnp.int32),
        pltpu.VMEM((ROWS_PER_W, HIST_PAD), jnp.int32),
        pltpu.VMEM((RING, IDX_PER_DMA, DIM), jnp.float32),
        pltpu.VMEM((ROWS_PER_W, DIM), jnp.float32),
        pltpu.VMEM((64,), jnp.float32),
        pltpu.SemaphoreType.DMA,
        pltpu.SemaphoreType.DMA,
        pltpu.SemaphoreType.DMA,
        pltpu.SemaphoreType.DMA,
    ],
)(_emb_bag_body)


def kernel(input, W):
    idx = input.astype(jnp.int32)
    idx_groups = idx.reshape(BATCH // GROUP, IDX_PER_DMA)
    idx_cnt = jnp.pad(idx, ((0, 0), (0, HIST_PAD - HIST)))
    rtab = jnp.asarray(_RSQRT_TAB)
    w_t3 = jnp.transpose(W).reshape(4, 8, NUM_EMB)
    w_tail = W[FULL_CHUNKS * CHUNK:].reshape(-1)
    w_flat = _transpose(w_t3, w_tail)
    w_lin = w_flat.reshape(NUM_EMB, DIM)
    return _emb_bag(idx_groups, idx_cnt, w_lin, rtab)


# speedup vs baseline: 1.9035x; 1.9035x over previous
import functools

import numpy as np
import jax
import jax.numpy as jnp
from jax import lax
from jax.experimental import pallas as pl
from jax.experimental.pallas import tpu as pltpu
from jax.experimental.pallas import tpu_sc as plsc

NUM_CORES = 2
NUM_SUBCORES = 16
NW = NUM_CORES * NUM_SUBCORES

BATCH = 16384
NUM_EMB = 1000000
HIST = 50
HIST_PAD = 56
GROUP = 4
IDX_PER_DMA = HIST * GROUP
DIM = 32
RING = 4

ROWS_PER_W = BATCH // NW
GROUPS_PER_W = ROWS_PER_W // GROUP

CHUNK = 128
FULL_CHUNKS = NUM_EMB // CHUNK
TAIL = NUM_EMB - FULL_CHUNKS * CHUNK
BASE_CHUNKS = FULL_CHUNKS // NW
EXTRA = FULL_CHUNKS - BASE_CHUNKS * NW

_RSQRT_TAB = np.zeros((64,), np.float32)
_RSQRT_TAB[: HIST + 1] = (
    1.0 / np.sqrt(np.maximum(np.arange(HIST + 1, dtype=np.float64), 1.0))
).astype(np.float32)


def _transpose_body(w_t3_hbm, w_tail_hbm, out_hbm,
                    tbuf0, tbuf1, obuf0, obuf1, tail_v, si0, si1, so0, so1):
    tbufs = (tbuf0, tbuf1)
    obufs = (obuf0, obuf1)
    sin = (si0, si1)
    sout = (so0, so1)
    wid = lax.axis_index("s") * NUM_CORES + lax.axis_index("c")
    n_w = BASE_CHUNKS + (wid < EXTRA).astype(jnp.int32)
    lo_w = wid * BASE_CHUNKS + jnp.minimum(wid, EXTRA)

    @pl.when(wid == 0)
    def _():
        pltpu.sync_copy(w_tail_hbm, tail_v)
        pltpu.sync_copy(tail_v, out_hbm.at[pl.ds(FULL_CHUNKS * CHUNK * DIM,
                                                 TAIL * DIM)])

    def start_in(c, s):
        pltpu.make_async_copy(
            w_t3_hbm.at[:, :, pl.ds(c * CHUNK, CHUNK)], tbufs[s], sin[s]
        ).start()

    def wait_in(s):
        pltpu.make_async_copy(
            w_t3_hbm.at[:, :, pl.ds(0, CHUNK)], tbufs[s], sin[s]
        ).wait()

    def start_out(c, s):
        pltpu.make_async_copy(
            obufs[s], out_hbm.at[pl.ds(c * CHUNK * DIM, CHUNK * DIM)],
            sout[s]
        ).start()

    def wait_out(s):
        pltpu.make_async_copy(
            obufs[s], out_hbm.at[pl.ds(0, CHUNK * DIM)], sout[s]
        ).wait()

    iota = lax.iota(jnp.int32, 16)

    def transpose_chunk(s):
        def dg_body(dg, car):
            m = (iota + dg) & 15
            rb_lo = lax.shift_right_logical(m, 3)
            dd = m & 7
            dst0 = iota * DIM + m
            for v0 in range(0, CHUNK, 16):
                v_vec = iota + v0
                for f0 in (0, 16):
                    rb_vec = rb_lo + (f0 // 8)
                    vec = plsc.load_gather(tbufs[s], [rb_vec, dd, v_vec])
                    plsc.store_scatter(
                        obufs[s], [dst0 + (v0 * DIM + f0)], vec
                    )
            return car

        lax.fori_loop(0, 16, dg_body, 0)

    start_in(lo_w, 0)
    start_in(lo_w + 1, 1)

    def loop_body(i, carry):
        for s in range(2):
            k = 2 * i + s
            c = lo_w + k

            @pl.when(k < n_w)
            def _():
                wait_in(s)

                @pl.when(k >= 2)
                def _():
                    wait_out(s)

                transpose_chunk(s)
                start_out(c, s)

                @pl.when(k + 2 < n_w)
                def _():
                    start_in(c + 2, s)

        return carry

    lax.fori_loop(0, (BASE_CHUNKS + 2) // 2, loop_body, 0)

    wait_out(0)
    wait_out(1)


_transpose = functools.partial(
    pl.kernel,
    out_type=jax.ShapeDtypeStruct((NUM_EMB * DIM,), jnp.float32),
    mesh=plsc.VectorSubcoreMesh(core_axis_name="c", subcore_axis_name="s"),
    compiler_params=pltpu.CompilerParams(
        use_tc_tiling_on_sc=True, needs_layout_passes=False
    ),
    scratch_types=[
        pltpu.VMEM((4, 8, CHUNK), jnp.float32),
        pltpu.VMEM((4, 8, CHUNK), jnp.float32),
        pltpu.VMEM((CHUNK * DIM,), jnp.float32),
        pltpu.VMEM((CHUNK * DIM,), jnp.float32),
        pltpu.VMEM((TAIL * DIM,), jnp.float32),
        pltpu.SemaphoreType.DMA,
        pltpu.SemaphoreType.DMA,
        pltpu.SemaphoreType.DMA,
        pltpu.SemaphoreType.DMA,
    ],
)(_transpose_body)


def _emb_bag_body(idx_g_hbm, idx_c_hbm, table_hbm, rtab_hbm, out_hbm,
                  idx_v, idx_c_v, gbuf, out_v, rtab_v, s0, s1, s2, s3):
    sems = (s0, s1, s2, s3)
    wid = lax.axis_index("s") * NUM_CORES + lax.axis_index("c")
    group_base = wid * GROUPS_PER_W
    row_base = wid * ROWS_PER_W

    pltpu.sync_copy(idx_g_hbm.at[pl.ds(group_base, GROUPS_PER_W)], idx_v)
    pltpu.sync_copy(idx_c_hbm.at[pl.ds(row_base, ROWS_PER_W)], idx_c_v)
    pltpu.sync_copy(rtab_hbm, rtab_v)

    def start(p, b):
        pltpu.make_async_copy(
            table_hbm.at[idx_v.at[p]], gbuf.at[b], sems[b]
        ).start()

    def wait(b):
        pltpu.make_async_copy(
            table_hbm.at[idx_v.at[0]], gbuf.at[b], sems[b]
        ).wait()

    def compute(p, b):
        for r in range(GROUP):
            out_row = p * GROUP + r
            i0 = idx_c_v[out_row, pl.ds(0, 16)]
            i1 = idx_c_v[out_row, pl.ds(16, 16)]
            i2 = idx_c_v[out_row, pl.ds(32, 16)]
            i3 = idx_c_v[out_row, pl.ds(40, 16)]
            tail_m = lax.iota(jnp.int32, 16) >= 8
            c_vec = (
                plsc.all_reduce_population_count(i0 != 0)
                + plsc.all_reduce_population_count(i1 != 0)
                + plsc.all_reduce_population_count(i2 != 0)
                + plsc.all_reduce_population_count((i3 != 0) & tail_m)
            )
            scale = plsc.load_gather(rtab_v, [c_vec])

            acc = [jnp.zeros((16,), jnp.float32) for _ in range(4)]
            for l in range(HIST):
                row = r * HIST + l
                j = l & 1
                acc[j] = acc[j] + gbuf[b, row, pl.ds(0, 16)]
                acc[2 + j] = acc[2 + j] + gbuf[b, row, pl.ds(16, 16)]

            out_v[out_row, pl.ds(0, 16)] = (acc[0] + acc[1]) * scale
            out_v[out_row, pl.ds(16, 16)] = (acc[2] + acc[3]) * scale

    for b in range(RING):
        start(b, b)

    def loop_body(i, carry):
        p0 = i * RING
        for b in range(RING):
            p = p0 + b
            wait(b)
            compute(p, b)

            @pl.when(p + RING < GROUPS_PER_W)
            def _():
                start(p + RING, b)

        return carry

    lax.fori_loop(0, GROUPS_PER_W // RING, loop_body, 0)

    pltpu.sync_copy(out_v, out_hbm.at[pl.ds(row_base, ROWS_PER_W)])


_emb_bag = functools.partial(
    pl.kernel,
    out_type=jax.ShapeDtypeStruct((BATCH, DIM), jnp.float32),
    mesh=plsc.VectorSubcoreMesh(core_axis_name="c", subcore_axis_name="s"),
    compiler_params=pltpu.CompilerParams(
        use_tc_tiling_on_sc=False, needs_layout_passes=False
    ),
    scratch_types=[
        pltpu.VMEM((GROUPS_PER_W, IDX_PER_DMA), jnp.int32),
        pltpu.VMEM((ROWS_PER_W, HIST_PAD), jnp.int32),
        pltpu.VMEM((RING, IDX_PER_DMA, DIM), jnp.float32),
        pltpu.VMEM((ROWS_PER_W, DIM), jnp.float32),
        pltpu.VMEM((64,), jnp.float32),
        pltpu.SemaphoreType.DMA,
        pltpu.SemaphoreType.DMA,
        pltpu.SemaphoreType.DMA,
        pltpu.SemaphoreType.DMA,
    ],
)(_emb_bag_body)


def kernel(input, W):
    idx = input.astype(jnp.int32)
    idx_groups = idx.reshape(BATCH // GROUP, IDX_PER_DMA)
    idx_cnt = jnp.pad(idx, ((0, 0), (0, HIST_PAD - HIST)))
    rtab = jnp.asarray(_RSQRT_TAB)
    w_t3 = jnp.transpose(W).reshape(4, 8, NUM_EMB)
    w_tail = W[FULL_CHUNKS * CHUNK:].reshape(-1)
    w_flat = _transpose(w_t3, w_tail)
    w_lin = w_flat.reshape(NUM_EMB, DIM)
    return _emb_bag(idx_groups, idx_cnt, w_lin, rtab)


# speedup vs baseline: 1.9052x; 1.0009x over previous
import functools

import numpy as np
import jax
import jax.numpy as jnp
from jax import lax
from jax.experimental import pallas as pl
from jax.experimental.pallas import tpu as pltpu
from jax.experimental.pallas import tpu_sc as plsc

NUM_CORES = 2
NUM_SUBCORES = 16
NW = NUM_CORES * NUM_SUBCORES

BATCH = 16384
NUM_EMB = 1000000
HIST = 50
HIST_PAD = 56
GROUP = 4
IDX_PER_DMA = HIST * GROUP
DIM = 32
RING = 4

ROWS_PER_W = BATCH // NW
GROUPS_PER_W = ROWS_PER_W // GROUP

CHUNK = 128
FULL_CHUNKS = NUM_EMB // CHUNK
TAIL = NUM_EMB - FULL_CHUNKS * CHUNK
BASE_CHUNKS = FULL_CHUNKS // NW
EXTRA = FULL_CHUNKS - BASE_CHUNKS * NW

_RSQRT_TAB = np.zeros((64,), np.float32)
_RSQRT_TAB[: HIST + 1] = (
    1.0 / np.sqrt(np.maximum(np.arange(HIST + 1, dtype=np.float64), 1.0))
).astype(np.float32)


def _transpose_body(w_t2_hbm, w_tail_hbm, out_hbm,
                    tbuf0, tbuf1, obuf0, obuf1, tail_v, si0, si1, so0, so1):
    tbufs = (tbuf0, tbuf1)
    obufs = (obuf0, obuf1)
    sin = (si0, si1)
    sout = (so0, so1)
    wid = lax.axis_index("s") * NUM_CORES + lax.axis_index("c")
    n_w = BASE_CHUNKS + (wid < EXTRA).astype(jnp.int32)
    lo_w = wid * BASE_CHUNKS + jnp.minimum(wid, EXTRA)

    @pl.when(wid == 0)
    def _():
        pltpu.sync_copy(w_tail_hbm, tail_v)
        pltpu.sync_copy(tail_v, out_hbm.at[pl.ds(FULL_CHUNKS * CHUNK * DIM,
                                                 TAIL * DIM)])

    def start_in(c, s):
        pltpu.make_async_copy(
            w_t2_hbm.at[:, pl.ds(c * CHUNK, CHUNK)], tbufs[s], sin[s]
        ).start()

    def wait_in(s):
        pltpu.make_async_copy(
            w_t2_hbm.at[:, pl.ds(0, CHUNK)], tbufs[s], sin[s]
        ).wait()

    def start_out(c, s):
        pltpu.make_async_copy(
            obufs[s], out_hbm.at[pl.ds(c * CHUNK * DIM, CHUNK * DIM)],
            sout[s]
        ).start()

    def wait_out(s):
        pltpu.make_async_copy(
            obufs[s], out_hbm.at[pl.ds(0, CHUNK * DIM)], sout[s]
        ).wait()

    iota = lax.iota(jnp.int32, 16)

    v_vecs = [iota + v0 for v0 in range(0, CHUNK, 16)]

    def transpose_chunk(s):
        def dg_body(dg, car):
            m = (iota + dg) & 15
            dst0 = iota * DIM + m
            for f0 in (0, 16):
                f_vec = m + f0
                for v0 in range(0, CHUNK, 16):
                    vec = plsc.load_gather(tbufs[s], [f_vec, v_vecs[v0 // 16]])
                    plsc.store_scatter(
                        obufs[s], [dst0 + (v0 * DIM + f0)], vec
                    )
            return car

        lax.fori_loop(0, 16, dg_body, 0)

    start_in(lo_w, 0)
    start_in(lo_w + 1, 1)

    def loop_body(i, carry):
        for s in range(2):
            k = 2 * i + s
            c = lo_w + k

            @pl.when(k < n_w)
            def _():
                wait_in(s)

                @pl.when(k >= 2)
                def _():
                    wait_out(s)

                transpose_chunk(s)
                start_out(c, s)

                @pl.when(k + 2 < n_w)
                def _():
                    start_in(c + 2, s)

        return carry

    lax.fori_loop(0, (BASE_CHUNKS + 2) // 2, loop_body, 0)

    wait_out(0)
    wait_out(1)


_transpose = functools.partial(
    pl.kernel,
    out_type=jax.ShapeDtypeStruct((NUM_EMB * DIM,), jnp.float32),
    mesh=plsc.VectorSubcoreMesh(core_axis_name="c", subcore_axis_name="s"),
    compiler_params=pltpu.CompilerParams(
        use_tc_tiling_on_sc=True, needs_layout_passes=False
    ),
    scratch_types=[
        pltpu.VMEM((32, CHUNK), jnp.float32),
        pltpu.VMEM((32, CHUNK), jnp.float32),
        pltpu.VMEM((CHUNK * DIM,), jnp.float32),
        pltpu.VMEM((CHUNK * DIM,), jnp.float32),
        pltpu.VMEM((TAIL * DIM,), jnp.float32),
        pltpu.SemaphoreType.DMA,
        pltpu.SemaphoreType.DMA,
        pltpu.SemaphoreType.DMA,
        pltpu.SemaphoreType.DMA,
    ],
)(_transpose_body)


def _emb_bag_body(idx_g_hbm, idx_c_hbm, table_hbm, rtab_hbm, out_hbm,
                  idx_v, idx_c_v, gbuf, out_v, rtab_v, s0, s1, s2, s3):
    sems = (s0, s1, s2, s3)
    wid = lax.axis_index("s") * NUM_CORES + lax.axis_index("c")
    group_base = wid * GROUPS_PER_W
    row_base = wid * ROWS_PER_W

    pltpu.sync_copy(idx_g_hbm.at[pl.ds(group_base, GROUPS_PER_W)], idx_v)
    pltpu.sync_copy(idx_c_hbm.at[pl.ds(row_base, ROWS_PER_W)], idx_c_v)
    pltpu.sync_copy(rtab_hbm, rtab_v)

    def start(p, b):
        pltpu.make_async_copy(
            table_hbm.at[idx_v.at[p]], gbuf.at[b], sems[b]
        ).start()

    def wait(b):
        pltpu.make_async_copy(
            table_hbm.at[idx_v.at[0]], gbuf.at[b], sems[b]
        ).wait()

    def compute(p, b):
        for r in range(GROUP):
            out_row = p * GROUP + r
            i0 = idx_c_v[out_row, pl.ds(0, 16)]
            i1 = idx_c_v[out_row, pl.ds(16, 16)]
            i2 = idx_c_v[out_row, pl.ds(32, 16)]
            i3 = idx_c_v[out_row, pl.ds(40, 16)]
            tail_m = lax.iota(jnp.int32, 16) >= 8
            c_vec = (
                plsc.all_reduce_population_count(i0 != 0)
                + plsc.all_reduce_population_count(i1 != 0)
                + plsc.all_reduce_population_count(i2 != 0)
                + plsc.all_reduce_population_count((i3 != 0) & tail_m)
            )
            scale = plsc.load_gather(rtab_v, [c_vec])

            acc = [jnp.zeros((16,), jnp.float32) for _ in range(4)]
            for l in range(HIST):
                row = r * HIST + l
                j = l & 1
                acc[j] = acc[j] + gbuf[b, row, pl.ds(0, 16)]
                acc[2 + j] = acc[2 + j] + gbuf[b, row, pl.ds(16, 16)]

            out_v[out_row, pl.ds(0, 16)] = (acc[0] + acc[1]) * scale
            out_v[out_row, pl.ds(16, 16)] = (acc[2] + acc[3]) * scale

    for b in range(RING):
        start(b, b)

    def loop_body(i, carry):
        p0 = i * RING
        for b in range(RING):
            p = p0 + b
            wait(b)
            compute(p, b)

            @pl.when(p + RING < GROUPS_PER_W)
            def _():
                start(p + RING, b)

        return carry

    lax.fori_loop(0, GROUPS_PER_W // RING, loop_body, 0)

    pltpu.sync_copy(out_v, out_hbm.at[pl.ds(row_base, ROWS_PER_W)])


_emb_bag = functools.partial(
    pl.kernel,
    out_type=jax.ShapeDtypeStruct((BATCH, DIM), jnp.float32),
    mesh=plsc.VectorSubcoreMesh(core_axis_name="c", subcore_axis_name="s"),
    compiler_params=pltpu.CompilerParams(
        use_tc_tiling_on_sc=False, needs_layout_passes=False
    ),
    scratch_types=[
        pltpu.VMEM((GROUPS_PER_W, IDX_PER_DMA), jnp.int32),
        pltpu.VMEM((ROWS_PER_W, HIST_PAD), jnp.int32),
        pltpu.VMEM((RING, IDX_PER_DMA, DIM), jnp.float32),
        pltpu.VMEM((ROWS_PER_W, DIM), jnp.float32),
        pltpu.VMEM((64,), jnp.float32),
        pltpu.SemaphoreType.DMA,
        pltpu.SemaphoreType.DMA,
        pltpu.SemaphoreType.DMA,
        pltpu.SemaphoreType.DMA,
    ],
)(_emb_bag_body)


def kernel(input, W):
    idx = input.astype(jnp.int32)
    idx_groups = idx.reshape(BATCH // GROUP, IDX_PER_DMA)
    idx_cnt = jnp.pad(idx, ((0, 0), (0, HIST_PAD - HIST)))
    rtab = jnp.asarray(_RSQRT_TAB)
    w_t2 = jnp.transpose(W)
    w_tail = W[FULL_CHUNKS * CHUNK:].reshape(-1)
    w_flat = _transpose(w_t2, w_tail)
    w_lin = w_flat.reshape(NUM_EMB, DIM)
    return _emb_bag(idx_groups, idx_cnt, w_lin, rtab)


# speedup vs baseline: 1.9321x; 1.0141x over previous
import functools

import numpy as np
import jax
import jax.numpy as jnp
from jax import lax
from jax.experimental import pallas as pl
from jax.experimental.pallas import tpu as pltpu
from jax.experimental.pallas import tpu_sc as plsc

NUM_CORES = 2
NUM_SUBCORES = 16
NW = NUM_CORES * NUM_SUBCORES

BATCH = 16384
NUM_EMB = 1000000
HIST = 50
HIST_PAD = 56
GROUP = 4
IDX_PER_DMA = HIST * GROUP
DIM = 32
RING = 4

ROWS_PER_W = BATCH // NW
GROUPS_PER_W = ROWS_PER_W // GROUP

CHUNK = 128
FULL_CHUNKS = NUM_EMB // CHUNK
TAIL = NUM_EMB - FULL_CHUNKS * CHUNK
BASE_CHUNKS = FULL_CHUNKS // NW
EXTRA = FULL_CHUNKS - BASE_CHUNKS * NW

_RSQRT_TAB = np.zeros((64,), np.float32)
_RSQRT_TAB[: HIST + 1] = (
    1.0 / np.sqrt(np.maximum(np.arange(HIST + 1, dtype=np.float64), 1.0))
).astype(np.float32)


def _transpose_body(w_t2_hbm, w_tail_hbm, out_hbm,
                    tbuf0, tbuf1, obuf0, obuf1, tail_v, si0, si1, so0, so1):
    tbufs = (tbuf0, tbuf1)
    obufs = (obuf0, obuf1)
    sin = (si0, si1)
    sout = (so0, so1)
    wid = lax.axis_index("s") * NUM_CORES + lax.axis_index("c")
    n_w = BASE_CHUNKS + (wid < EXTRA).astype(jnp.int32)
    lo_w = wid * BASE_CHUNKS + jnp.minimum(wid, EXTRA)

    @pl.when(wid == 0)
    def _():
        pltpu.sync_copy(w_tail_hbm, tail_v)
        pltpu.sync_copy(tail_v, out_hbm.at[pl.ds(FULL_CHUNKS * CHUNK * DIM,
                                                 TAIL * DIM)])

    def start_in(c, s):
        pltpu.make_async_copy(
            w_t2_hbm.at[:, pl.ds(c * CHUNK, CHUNK)], tbufs[s], sin[s]
        ).start()

    def wait_in(s):
        pltpu.make_async_copy(
            w_t2_hbm.at[:, pl.ds(0, CHUNK)], tbufs[s], sin[s]
        ).wait()

    def start_out(c, s):
        pltpu.make_async_copy(
            obufs[s], out_hbm.at[pl.ds(c * CHUNK * DIM, CHUNK * DIM)],
            sout[s]
        ).start()

    def wait_out(s):
        pltpu.make_async_copy(
            obufs[s], out_hbm.at[pl.ds(0, CHUNK * DIM)], sout[s]
        ).wait()

    iota = lax.iota(jnp.int32, 16)

    def transpose_chunk(s):
        def dg_body(i, car):
            for u in range(2):
                dg = i * 2 + u
                m = (iota + dg) & 15
                dst0 = iota * DIM + m
                for f0 in (0, 16):
                    f_vec = m + f0
                    for v0 in range(0, CHUNK, 16):
                        vec = plsc.load_gather(
                            tbufs[s], [f_vec, iota + v0]
                        )
                        plsc.store_scatter(
                            obufs[s], [dst0 + (v0 * DIM + f0)], vec
                        )
            return car

        lax.fori_loop(0, 8, dg_body, 0)

    start_in(lo_w, 0)
    start_in(lo_w + 1, 1)

    def loop_body(i, carry):
        for s in range(2):
            k = 2 * i + s
            c = lo_w + k

            @pl.when(k < n_w)
            def _():
                wait_in(s)

                @pl.when(k >= 2)
                def _():
                    wait_out(s)

                transpose_chunk(s)
                start_out(c, s)

                @pl.when(k + 2 < n_w)
                def _():
                    start_in(c + 2, s)

        return carry

    lax.fori_loop(0, (BASE_CHUNKS + 2) // 2, loop_body, 0)

    wait_out(0)
    wait_out(1)


_transpose = functools.partial(
    pl.kernel,
    out_type=jax.ShapeDtypeStruct((NUM_EMB * DIM,), jnp.float32),
    mesh=plsc.VectorSubcoreMesh(core_axis_name="c", subcore_axis_name="s"),
    compiler_params=pltpu.CompilerParams(
        use_tc_tiling_on_sc=True, needs_layout_passes=False
    ),
    scratch_types=[
        pltpu.VMEM((32, CHUNK), jnp.float32),
        pltpu.VMEM((32, CHUNK), jnp.float32),
        pltpu.VMEM((CHUNK * DIM,), jnp.float32),
        pltpu.VMEM((CHUNK * DIM,), jnp.float32),
        pltpu.VMEM((TAIL * DIM,), jnp.float32),
        pltpu.SemaphoreType.DMA,
        pltpu.SemaphoreType.DMA,
        pltpu.SemaphoreType.DMA,
        pltpu.SemaphoreType.DMA,
    ],
)(_transpose_body)


def _emb_bag_body(idx_g_hbm, idx_c_hbm, table_hbm, rtab_hbm, out_hbm,
                  idx_v, idx_c_v, gbuf, out_v, rtab_v, s0, s1, s2, s3):
    sems = (s0, s1, s2, s3)
    wid = lax.axis_index("s") * NUM_CORES + lax.axis_index("c")
    group_base = wid * GROUPS_PER_W
    row_base = wid * ROWS_PER_W

    pltpu.sync_copy(idx_g_hbm.at[pl.ds(group_base, GROUPS_PER_W)], idx_v)
    pltpu.sync_copy(idx_c_hbm.at[pl.ds(row_base, ROWS_PER_W)], idx_c_v)
    pltpu.sync_copy(rtab_hbm, rtab_v)

    def start(p, b):
        pltpu.make_async_copy(
            table_hbm.at[idx_v.at[p]], gbuf.at[b], sems[b]
        ).start()

    def wait(b):
        pltpu.make_async_copy(
            table_hbm.at[idx_v.at[0]], gbuf.at[b], sems[b]
        ).wait()

    def compute(p, b):
        for r in range(GROUP):
            out_row = p * GROUP + r
            i0 = idx_c_v[out_row, pl.ds(0, 16)]
            i1 = idx_c_v[out_row, pl.ds(16, 16)]
            i2 = idx_c_v[out_row, pl.ds(32, 16)]
            i3 = idx_c_v[out_row, pl.ds(40, 16)]
            tail_m = lax.iota(jnp.int32, 16) >= 8
            c_vec = (
                plsc.all_reduce_population_count(i0 != 0)
                + plsc.all_reduce_population_count(i1 != 0)
                + plsc.all_reduce_population_count(i2 != 0)
                + plsc.all_reduce_population_count((i3 != 0) & tail_m)
            )
            scale = plsc.load_gather(rtab_v, [c_vec])

            acc = [jnp.zeros((16,), jnp.float32) for _ in range(4)]
            for l in range(HIST):
                row = r * HIST + l
                j = l & 1
                acc[j] = acc[j] + gbuf[b, row, pl.ds(0, 16)]
                acc[2 + j] = acc[2 + j] + gbuf[b, row, pl.ds(16, 16)]

            out_v[out_row, pl.ds(0, 16)] = (acc[0] + acc[1]) * scale
            out_v[out_row, pl.ds(16, 16)] = (acc[2] + acc[3]) * scale

    for b in range(RING):
        start(b, b)

    def loop_body(i, carry):
        p0 = i * RING
        for b in range(RING):
            p = p0 + b
            wait(b)
            compute(p, b)

            @pl.when(p + RING < GROUPS_PER_W)
            def _():
                start(p + RING, b)

        return carry

    lax.fori_loop(0, GROUPS_PER_W // RING, loop_body, 0)

    pltpu.sync_copy(out_v, out_hbm.at[pl.ds(row_base, ROWS_PER_W)])


_emb_bag = functools.partial(
    pl.kernel,
    out_type=jax.ShapeDtypeStruct((BATCH, DIM), jnp.float32),
    mesh=plsc.VectorSubcoreMesh(core_axis_name="c", subcore_axis_name="s"),
    compiler_params=pltpu.CompilerParams(
        use_tc_tiling_on_sc=False, needs_layout_passes=False
    ),
    scratch_types=[
        pltpu.VMEM((GROUPS_PER_W, IDX_PER_DMA), jnp.int32),
        pltpu.VMEM((ROWS_PER_W, HIST_PAD), jnp.int32),
        pltpu.VMEM((RING, IDX_PER_DMA, DIM), jnp.float32),
        pltpu.VMEM((ROWS_PER_W, DIM), jnp.float32),
        pltpu.VMEM((64,), jnp.float32),
        pltpu.SemaphoreType.DMA,
        pltpu.SemaphoreType.DMA,
        pltpu.SemaphoreType.DMA,
        pltpu.SemaphoreType.DMA,
    ],
)(_emb_bag_body)


def kernel(input, W):
    idx = input.astype(jnp.int32)
    idx_groups = idx.reshape(BATCH // GROUP, IDX_PER_DMA)
    idx_cnt = jnp.pad(idx, ((0, 0), (0, HIST_PAD - HIST)))
    rtab = jnp.asarray(_RSQRT_TAB)
    w_t2 = jnp.transpose(W)
    w_tail = W[FULL_CHUNKS * CHUNK:].reshape(-1)
    w_flat = _transpose(w_t2, w_tail)
    w_lin = w_flat.reshape(NUM_EMB, DIM)
    return _emb_bag(idx_groups, idx_cnt, w_lin, rtab)
